# baseline (speedup 1.0000x reference)
import functools

import jax
import jax.numpy as jnp
from jax import lax
from jax.experimental import pallas as pl
from jax.experimental.pallas import tpu as pltpu
from jax.experimental.pallas import tpu_sc as plsc

B = 4
H = W = 224
HF = WF = 112
NPIX = HF * WF
NTOT = B * NPIX
NSEG = 256
NTOK = B * NSEG
CE = 768
CHID = 512
C1 = 64
C1P = 128
K1 = 147
K1P = 152
PIXT = 1792
EPS = 1e-5


def _k1_body(p_ref, w_ref, b_ref, g_ref, bb_ref, raw_ref, st_ref):
    bi = pl.program_id(0)
    ti = pl.program_id(1)
    x = p_ref[0]
    acc = jnp.dot(w_ref[...], x, preferred_element_type=jnp.float32)
    acc = acc + b_ref[...]
    raw_ref[0] = acc

    @pl.when((bi == 0) & (ti == 0))
    def _():
        st_ref[...] = jnp.zeros_like(st_ref)

    st_ref[:, 0:1] += jnp.sum(acc, axis=1, keepdims=True)
    st_ref[:, 1:2] += jnp.sum(acc * acc, axis=1, keepdims=True)

    @pl.when((bi == B - 1) & (ti == NPIX // PIXT - 1))
    def _():
        mean = st_ref[:, 0:1] / NTOT
        var = st_ref[:, 1:2] / NTOT - mean * mean
        scale = g_ref[...] * lax.rsqrt(var + EPS)
        shift = bb_ref[...] - mean * scale
        st_ref[:, 2:3] = scale
        st_ref[:, 3:4] = shift


def _k1_call(patches, w1, b1, g1, bb1, interpret=False):
    return pl.pallas_call(
        _k1_body,
        grid=(B, NPIX // PIXT),
        in_specs=[
            pl.BlockSpec((1, K1P, PIXT), lambda b, t: (b, 0, t)),
            pl.BlockSpec((C1P, K1P), lambda b, t: (0, 0)),
            pl.BlockSpec((C1P, 1), lambda b, t: (0, 0)),
            pl.BlockSpec((C1P, 1), lambda b, t: (0, 0)),
            pl.BlockSpec((C1P, 1), lambda b, t: (0, 0)),
        ],
        out_specs=[
            pl.BlockSpec((1, C1P, PIXT), lambda b, t: (b, 0, t)),
            pl.BlockSpec((C1P, 8), lambda b, t: (0, 0)),
        ],
        out_shape=[
            jax.ShapeDtypeStruct((B, C1P, NPIX), jnp.float32),
            jax.ShapeDtypeStruct((C1P, 8), jnp.float32),
        ],
        interpret=interpret,
    )(patches, w1, b1, g1, bb1)


def _k1b_body(raw_ref, st_ref, out_ref):
    scale = st_ref[:, 2:3]
    shift = st_ref[:, 3:4]
    out_ref[0] = jnp.maximum(raw_ref[0] * scale + shift, 0.0)


def _k1b_call(raw1, st1, interpret=False):
    return pl.pallas_call(
        _k1b_body,
        grid=(B, NPIX // PIXT),
        in_specs=[
            pl.BlockSpec((1, C1P, PIXT), lambda b, t: (b, 0, t)),
            pl.BlockSpec((C1P, 8), lambda b, t: (0, 0)),
        ],
        out_specs=pl.BlockSpec((1, C1P, PIXT), lambda b, t: (b, 0, t)),
        out_shape=jax.ShapeDtypeStruct((B, C1P, NPIX), jnp.float32),
        interpret=interpret,
    )(raw1, st1)


ROWT = 8


def _k2_body(f_ref, w_ref, b_ref, g_ref, bb_ref, raw_ref, st_ref):
    bi = pl.program_id(0)
    ti = pl.program_id(1)

    @pl.when((bi == 0) & (ti == 0))
    def _():
        st_ref[...] = jnp.zeros_like(st_ref)

    rs = jnp.zeros((CE, 1), jnp.float32)
    rq = jnp.zeros((CE, 1), jnp.float32)
    for r in range(ROWT):
        acc = jnp.broadcast_to(b_ref[...], (CE, WF)).astype(jnp.float32)
        for dy in range(3):
            xrow = f_ref[0, :, pl.ds(ti * ROWT + r + dy, 1), :]
            xrow = xrow.reshape(C1P, WF + 2)
            for dx in range(3):
                xk = xrow[:, dx:dx + WF]
                acc = acc + jnp.dot(w_ref[dy * 3 + dx], xk,
                                    preferred_element_type=jnp.float32)
        raw_ref[0, :, pl.ds(r, 1), :] = acc[:, None, :]
        rs = rs + jnp.sum(acc, axis=1, keepdims=True)
        rq = rq + jnp.sum(acc * acc, axis=1, keepdims=True)

    st_ref[:, 0:1] += rs
    st_ref[:, 1:2] += rq

    @pl.when((bi == B - 1) & (ti == HF // ROWT - 1))
    def _():
        mean = st_ref[:, 0:1] / NTOT
        var = st_ref[:, 1:2] / NTOT - mean * mean
        scale = g_ref[...] * lax.rsqrt(var + EPS)
        shift = bb_ref[...] - mean * scale
        st_ref[:, 2:3] = scale
        st_ref[:, 3:4] = shift


def _k2_call(f1p, w2, b2, g2, bb2, interpret=False):
    return pl.pallas_call(
        _k2_body,
        grid=(B, HF // ROWT),
        in_specs=[
            pl.BlockSpec((1, C1P, HF + 2, WF + 2), lambda b, t: (b, 0, 0, 0)),
            pl.BlockSpec((9, CE, C1P), lambda b, t: (0, 0, 0)),
            pl.BlockSpec((CE, 1), lambda b, t: (0, 0)),
            pl.BlockSpec((CE, 1), lambda b, t: (0, 0)),
            pl.BlockSpec((CE, 1), lambda b, t: (0, 0)),
        ],
        out_specs=[
            pl.BlockSpec((1, CE, ROWT, WF), lambda b, t: (b, 0, t, 0)),
            pl.BlockSpec((CE, 8), lambda b, t: (0, 0)),
        ],
        out_shape=[
            jax.ShapeDtypeStruct((B, CE, HF, WF), jnp.float32),
            jax.ShapeDtypeStruct((CE, 8), jnp.float32),
        ],
        interpret=interpret,
    )(f1p, w2, b2, g2, bb2)


SEGR = NPIX // 128


def _k3a_body(seg_ref, win_ref):
    ids = lax.broadcasted_iota(jnp.int32, (NSEG, 128), 0)
    for b in range(B):
        def body(r, acc):
            seg_row = seg_ref[b, pl.ds(r, 1), :]
            n_row = r * 128 + lax.broadcasted_iota(jnp.int32, (1, 128), 1)
            return jnp.where(seg_row == ids, n_row, acc)
        acc = lax.fori_loop(0, SEGR, body,
                            jnp.full((NSEG, 128), -1, jnp.int32))
        win_ref[b] = jnp.max(acc, axis=1, keepdims=True)


def _k3a_call(seg3, interpret=False):
    return pl.pallas_call(
        _k3a_body,
        out_shape=jax.ShapeDtypeStruct((B, NSEG, 1), jnp.int32),
        interpret=interpret,
    )(seg3)


def _sc_gather(table, win_flat):
    info = plsc.get_sparse_core_info()
    nc, ns = info.num_cores, info.num_subcores
    nw = nc * ns
    tpw = NTOK // nw
    mesh = plsc.VectorSubcoreMesh(core_axis_name="c", subcore_axis_name="s")

    @functools.partial(
        pl.kernel,
        out_type=jax.ShapeDtypeStruct((NTOK, CE), jnp.float32),
        mesh=mesh,
        scratch_types=[
            pltpu.VMEM((tpw,), jnp.int32),
            pltpu.VMEM((tpw, CE), jnp.float32),
            pltpu.SemaphoreType.DMA,
        ],
    )
    def k(win_hbm, table_hbm, out_hbm, idx_v, rows_v, sem):
        wid = lax.axis_index("s") * nc + lax.axis_index("c")
        base = wid * tpw
        pltpu.sync_copy(win_hbm.at[pl.ds(base, tpw)], idx_v)
        off = (base // NSEG) * NPIX
        for c in range(tpw // 16):
            v = idx_v[pl.ds(c * 16, 16)]
            idx_v[pl.ds(c * 16, 16)] = jnp.maximum(v, 0) + off
        pltpu.async_copy(table_hbm.at[idx_v], rows_v, sem).wait()
        pltpu.sync_copy(rows_v, out_hbm.at[pl.ds(base, tpw)])

    return k(win_flat, table)


def _k4_body(rows_ref, win_ref, sc2_ref, pos_ref, l1t_ref, l1b_ref,
             l2t_ref, l2b_ref, out_ref):
    g = rows_ref[...]
    win = win_ref[...]
    scale2 = sc2_ref[0:1, :]
    shift2 = sc2_ref[1:2, :]
    winf = win.astype(jnp.float32)
    ch0 = jnp.floor(3.0 * winf / 49.0 + 1e-4).astype(jnp.int32)
    ch1 = jnp.minimum(ch0 + 1, CE - 1)
    t = (ch0 + 1) * NPIX - win * CE
    jlane = lax.broadcasted_iota(jnp.int32, (1, CE), 1)
    a_s = jnp.sum(jnp.where(ch0 == jlane, scale2, 0.0), axis=1, keepdims=True)
    b_s = jnp.sum(jnp.where(ch1 == jlane, scale2, 0.0), axis=1, keepdims=True)
    a_h = jnp.sum(jnp.where(ch0 == jlane, shift2, 0.0), axis=1, keepdims=True)
    b_h = jnp.sum(jnp.where(ch1 == jlane, shift2, 0.0), axis=1, keepdims=True)
    in_first = jlane < t
    scale_m = jnp.where(in_first, a_s, b_s)
    shift_m = jnp.where(in_first, a_h, b_h)
    x = jnp.maximum(g * scale_m + shift_m, 0.0) + pos_ref[...]
    h = jnp.dot(x, l1t_ref[...], preferred_element_type=jnp.float32)
    h = h + l1b_ref[0:1, :]
    h = 0.5 * h * (1.0 + lax.erf(h * 0.7071067811865476))
    y = jnp.dot(h, l2t_ref[...], preferred_element_type=jnp.float32)
    y = y + l2b_ref[0:1, :]
    out_ref[...] = jnp.where(win >= 0, y, 0.0)


def _k4_call(rows, win2, sc2, pos, l1t, l1b, l2t, l2b, interpret=False):
    return pl.pallas_call(
        _k4_body,
        out_shape=jax.ShapeDtypeStruct((NTOK, CE), jnp.float32),
        interpret=interpret,
    )(rows, win2, sc2, pos, l1t, l1b, l2t, l2b)


def kernel(img, segments, conv1_w, conv1_b, bn1_g, bn1_b, conv2_w, conv2_b,
           bn2_g, bn2_b, pos_table, L1_w, L1_b, L2_w, L2_b):
    f32 = jnp.float32

    imgp = jnp.pad(img, ((0, 0), (0, 0), (3, 3), (3, 3)))
    sl = [imgp[:, :, dy:dy + 223:2, dx:dx + 223:2]
          for dy in range(7) for dx in range(7)]
    patches = jnp.stack(sl, axis=2).reshape(B, K1, NPIX)
    patches = jnp.pad(patches, ((0, 0), (0, K1P - K1), (0, 0)))

    w1 = jnp.pad(conv1_w.reshape(C1, K1), ((0, C1P - C1), (0, K1P - K1)))
    b1 = jnp.pad(conv1_b, (0, C1P - C1)).reshape(C1P, 1)
    g1 = jnp.pad(bn1_g, (0, C1P - C1)).reshape(C1P, 1)
    bb1 = jnp.pad(bn1_b, (0, C1P - C1)).reshape(C1P, 1)

    raw1, st1 = _k1_call(patches, w1, b1, g1, bb1)
    f1n = _k1b_call(raw1, st1)
    f1p = jnp.pad(f1n.reshape(B, C1P, HF, WF),
                  ((0, 0), (0, 0), (1, 1), (1, 1)))

    w2 = jnp.pad(conv2_w.transpose(2, 3, 0, 1).reshape(9, CE, C1),
                 ((0, 0), (0, 0), (0, C1P - C1)))
    b2 = conv2_b.reshape(CE, 1)
    g2 = bn2_g.reshape(CE, 1)
    bb2 = bn2_b.reshape(CE, 1)

    raw2, st2 = _k2_call(f1p, w2, b2, g2, bb2)

    seg3 = segments[:, ::2, ::2].reshape(B, SEGR, 128).astype(jnp.int32)
    win = _k3a_call(seg3)

    table = raw2.reshape(NTOT, CE)
    rows = _sc_gather(table, win.reshape(NTOK))

    sc2 = jnp.tile(jnp.stack([st2[:, 2], st2[:, 3]], axis=0), (4, 1))
    pos = jnp.tile(pos_table, (B, 1)).astype(f32)
    l1t = L1_w.T
    l1b = jnp.tile(L1_b.reshape(1, CHID), (8, 1))
    l2t = L2_w.T
    l2b = jnp.tile(L2_b.reshape(1, CE), (8, 1))

    y = _k4_call(rows, win.reshape(NTOK, 1), sc2, pos, l1t, l1b, l2t, l2b)
    return y.reshape(B, NSEG, CE)
